# Initial kernel scaffold; baseline (speedup 1.0000x reference)
import jax
import jax.numpy as jnp
from jax.experimental import pallas as pl


def kernel(z_e, embedding):
    raise NotImplementedError("write your pallas kernel here")


# speedup vs baseline: 1.0811x; 1.0811x over previous
import jax
import jax.numpy as jnp
from jax.experimental import pallas as pl
from jax.experimental.pallas import tpu as pltpu

K = 8192
D = 256
N = 8192
T = 256
GRID = N // T
COMMITMENT_COST = 0.25


def _vq_body(x_ref, emb_ref, st_ref, enc_ref, loss_ref, perp_ref,
             loss_acc, hist_acc):
    i = pl.program_id(0)
    x = x_ref[...]
    e = emb_ref[...]

    @pl.when(i == 0)
    def _init():
        loss_acc[0, 0] = jnp.float32(0.0)
        hist_acc[...] = jnp.zeros_like(hist_acc)

    x_norm = jnp.sum(x * x, axis=1, keepdims=True)
    e_norm = jnp.sum(e * e, axis=1)
    xb = x.astype(jnp.bfloat16)
    eb = e.astype(jnp.bfloat16)
    prod = jax.lax.dot_general(xb, eb, (((1,), (1,)), ((), ())),
                               preferred_element_type=jnp.float32)
    dist = (x_norm + e_norm[None, :]) - 2.0 * prod

    mn = jnp.min(dist, axis=1, keepdims=True)
    iota = jax.lax.broadcasted_iota(jnp.int32, (T, K), 1)
    idx = jnp.min(jnp.where(dist == mn, iota, K), axis=1)
    enc = (iota == idx[:, None]).astype(jnp.float32)
    enc_ref[...] = enc

    zq = jax.lax.dot_general(enc.astype(jnp.bfloat16), eb, (((1,), (0,)), ((), ())),
                             preferred_element_type=jnp.float32)
    t = zq - x
    st_ref[...] = x + t
    loss_acc[0, 0] += jnp.sum(t * t)
    hist_acc[...] += jnp.sum(enc, axis=0)[None, :]

    @pl.when(i == GRID - 1)
    def _fini():
        m = loss_acc[0, 0] / jnp.float32(N * D)
        loss_ref[0, 0] = m + COMMITMENT_COST * m
        avg = hist_acc[...] * jnp.float32(1.0 / N)
        ent = jnp.sum(avg * jnp.log(avg + 1e-10))
        perp_ref[0, 0] = jnp.exp(-ent)


def kernel(z_e, embedding):
    B, Dm, H, W = z_e.shape
    z = jnp.transpose(z_e, (0, 2, 3, 1)).reshape(N, D)
    st, enc, loss, perp = pl.pallas_call(
        _vq_body,
        grid=(GRID,),
        in_specs=[
            pl.BlockSpec((T, D), lambda i: (i, 0)),
            pl.BlockSpec((K, D), lambda i: (0, 0)),
        ],
        out_specs=[
            pl.BlockSpec((T, D), lambda i: (i, 0)),
            pl.BlockSpec((T, K), lambda i: (i, 0)),
            pl.BlockSpec((1, 1), lambda i: (0, 0), memory_space=pltpu.SMEM),
            pl.BlockSpec((1, 1), lambda i: (0, 0), memory_space=pltpu.SMEM),
        ],
        out_shape=[
            jax.ShapeDtypeStruct((N, D), jnp.float32),
            jax.ShapeDtypeStruct((N, K), jnp.float32),
            jax.ShapeDtypeStruct((1, 1), jnp.float32),
            jax.ShapeDtypeStruct((1, 1), jnp.float32),
        ],
        scratch_shapes=[
            pltpu.SMEM((1, 1), jnp.float32),
            pltpu.VMEM((1, K), jnp.float32),
        ],
    )(z, embedding)
    out = jnp.transpose(st.reshape(B, H, W, Dm), (0, 3, 1, 2))
    return out, loss[0, 0], perp[0, 0], enc
